# baseline (speedup 1.0000x reference)
import functools

import jax
import jax.numpy as jnp
from jax import lax
from jax.experimental import pallas as pl
from jax.experimental.pallas import tpu as pltpu
from jax.experimental.pallas import tpu_sc as plsc

_BATCH = 16384
_HIST = 20
_B = _BATCH * _HIST
_D = 32
_DW = _D // 2
_CHUNK = 128
_NBUF = 4
_NC = 2
_NS = 16


@functools.cache
def _build():
    nw = _NC * _NS
    b_per_w = _B // nw
    n_chunks = b_per_w // _CHUNK
    mesh = plsc.VectorSubcoreMesh(
        core_axis_name="c", subcore_axis_name="s",
        num_cores=_NC, num_subcores=_NS,
    )

    @functools.partial(
        pl.kernel,
        out_type=jax.ShapeDtypeStruct((_B, _DW), jnp.int32),
        mesh=mesh,
        scratch_types=[
            pltpu.VMEM((b_per_w,), jnp.int32),
            pltpu.VMEM((_NBUF, _CHUNK, _DW), jnp.int32),
        ] + [pltpu.SemaphoreType.DMA] * _NBUF,
        compiler_params=pltpu.CompilerParams(use_tc_tiling_on_sc=False),
    )
    def embed(idx_hbm, table_hbm, out_hbm, idx_v, rows_v, *sems):
        out2 = out_hbm

        wid = lax.axis_index("s") * _NC + lax.axis_index("c")
        row0 = wid * b_per_w

        pltpu.sync_copy(idx_hbm.at[pl.ds(row0, b_per_w)], idx_v)

        def issue(j, b):
            pltpu.async_copy(
                table_hbm.at[idx_v.at[pl.ds(j * _CHUNK, _CHUNK)]],
                rows_v.at[b], sems[b],
            )

        def wait(j, b):
            pltpu.make_async_copy(
                table_hbm.at[idx_v.at[pl.ds(j * _CHUNK, _CHUNK)]],
                rows_v.at[b], sems[b],
            ).wait()

        def store(j, b):
            pltpu.sync_copy(
                rows_v.at[b], out2.at[pl.ds(row0 + j * _CHUNK, _CHUNK)]
            )

        for b in range(_NBUF):
            issue(b, b)

        @pl.loop(0, n_chunks - _NBUF, step=_NBUF)
        def _(g):
            for b in range(_NBUF):
                j = g + b
                wait(j, b)
                store(j, b)
                issue(j + _NBUF, b)

        for b in range(_NBUF):
            j = n_chunks - _NBUF + b
            wait(j, b)
            store(j, b)

    return embed


def kernel(inputs, embedding):
    idx_flat = inputs.reshape(_B)
    table_i32 = lax.bitcast_convert_type(
        embedding.reshape(embedding.shape[0], _DW, 2), jnp.int32
    )
    out = _build()(idx_flat, table_i32)
    out_bf16 = lax.bitcast_convert_type(out, jnp.bfloat16)
    return out_bf16.reshape(_BATCH, _HIST, _D)


# speedup vs baseline: 2.4940x; 2.4940x over previous
import functools

import jax
import jax.numpy as jnp
from jax import lax
from jax.experimental import pallas as pl
from jax.experimental.pallas import tpu as pltpu
from jax.experimental.pallas import tpu_sc as plsc

_BATCH = 16384
_HIST = 20
_D = 32
_PW = _D // 2
_PACK = 8
_SCR_ROWS = 1000000 // _PACK
_SW = _PACK * _PW
_NBLK = 7813
_CHUNK = 128
_NC = 2
_NS = 16

_CP = pltpu.CompilerParams(use_tc_tiling_on_sc=True, needs_layout_passes=False)
_MESH = dict(core_axis_name="c", subcore_axis_name="s",
             num_cores=_NC, num_subcores=_NS)


@functools.cache
def _build_transpose():
    mesh = plsc.VectorSubcoreMesh(**_MESH)

    @functools.partial(
        pl.kernel,
        out_type=jax.ShapeDtypeStruct((_SCR_ROWS, _SW), jnp.int32),
        mesh=mesh,
        scratch_types=[
            pltpu.VMEM((_PW, 128), jnp.int32),
            pltpu.VMEM((_PW, 64), jnp.int32),
            pltpu.VMEM((_PW, 128), jnp.int32),
        ],
        compiler_params=_CP,
    )
    def transpose(table_hbm, tail_hbm, scr_hbm, blk_v, tblk_v, tp_v):
        tblw = table_hbm.bitcast(jnp.int32)
        wid = lax.axis_index("s") * _NC + lax.axis_index("c")

        def flip(src_v, width):
            @pl.loop(0, width)
            def _(r):
                col = plsc.load_gather(
                    src_v,
                    [lax.iota(jnp.int32, _PW),
                     jnp.zeros((_PW,), jnp.int32) + r],
                )
                tp_v[r >> 3, pl.ds((r & 7) * _PW, _PW)] = col

        def step(blk):
            pltpu.sync_copy(tblw.at[:, pl.ds(blk * 128, 128)], blk_v)
            flip(blk_v, 128)
            pltpu.sync_copy(tp_v, scr_hbm.at[pl.ds(blk * 16, _PW), :])

        @pl.loop(0, (_NBLK - 1 + 31) // 32)
        def _(i):
            blk = wid + 32 * i

            @pl.when(blk < _NBLK - 1)
            def _():
                step(blk)

        @pl.when(wid == 4)
        def _():
            pltpu.sync_copy(tail_hbm.bitcast(jnp.int32), tblk_v)
            flip(tblk_v, 64)
            pltpu.sync_copy(
                tp_v.at[pl.ds(0, 8), :],
                scr_hbm.at[pl.ds((_NBLK - 1) * 16, 8), :],
            )

    return transpose


@functools.cache
def _build_gather():
    nw = _NC * _NS
    b_per_w = _BATCH // nw
    n_sub = b_per_w // _CHUNK
    mesh = plsc.VectorSubcoreMesh(**_MESH)

    @functools.partial(
        pl.kernel,
        out_type=jax.ShapeDtypeStruct((_HIST, _D, _BATCH), jnp.bfloat16),
        mesh=mesh,
        scratch_types=[
            pltpu.VMEM((b_per_w,), jnp.int32),
            pltpu.VMEM((2, _CHUNK), jnp.int32),
            pltpu.VMEM((2, _CHUNK), jnp.int32),
            pltpu.VMEM((2, _CHUNK, _SW), jnp.int32),
            pltpu.VMEM((_PW, b_per_w), jnp.int32),
            pltpu.SemaphoreType.DMA,
            pltpu.SemaphoreType.DMA,
        ],
        compiler_params=_CP,
    )
    def gather(idx_hbm, scr_hbm, out_hbm, sidx_v, rid_v, off_v, pair_v,
               ob_v, sem0, sem1):
        wid = lax.axis_index("s") * _NC + lax.axis_index("c")
        b0 = wid * b_per_w
        sems = (sem0, sem1)

        def prep(c, tc):
            @pl.loop(0, _CHUNK, step=16)
            def _(i):
                v = sidx_v[pl.ds(c * _CHUNK + i, 16)]
                rid_v[tc, pl.ds(i, 16)] = v >> 3
                off_v[tc, pl.ds(i, 16)] = (v & 7) * _PW

        def issue(tc):
            pltpu.async_copy(scr_hbm.at[rid_v.at[tc]], pair_v.at[tc],
                             sems[tc])

        def drain(tc):
            pltpu.make_async_copy(
                scr_hbm.at[pl.ds(0, _CHUNK)], pair_v.at[tc], sems[tc]
            ).wait()

        def extract(c, tc):
            for kg in range(_CHUNK // 16):
                ks = lax.iota(jnp.int32, 16) + (kg * 16)
                offs = off_v[tc, pl.ds(kg * 16, 16)]

                @pl.loop(0, _PW)
                def _(p):
                    col = plsc.load_gather(pair_v.at[tc], [ks, offs + p])
                    ob_v[p, pl.ds(c * _CHUNK + kg * 16, 16)] = col

        @pl.loop(0, _HIST)
        def _(h):
            pltpu.sync_copy(idx_hbm.at[h, pl.ds(b0, b_per_w)], sidx_v)
            prep(0, 0)
            issue(0)
            for c in range(n_sub):
                tc = c % 2
                if c + 1 < n_sub:
                    prep(c + 1, 1 - tc)
                    issue(1 - tc)
                drain(tc)
                extract(c, tc)

            outw = out_hbm.at[h].bitcast(jnp.int32)
            pltpu.sync_copy(ob_v, outw.at[:, pl.ds(b0, b_per_w)])

    return gather


def kernel(inputs, embedding):
    table_t = embedding.T
    tail_t = table_t[:, (_NBLK - 1) * 128:]
    scr = _build_transpose()(table_t, tail_t)
    out = _build_gather()(inputs.T, scr)
    return out.transpose(2, 0, 1)


# speedup vs baseline: 3.3790x; 1.3548x over previous
import functools

import jax
import jax.numpy as jnp
from jax import lax
from jax.experimental import pallas as pl
from jax.experimental.pallas import tpu as pltpu
from jax.experimental.pallas import tpu_sc as plsc

_BATCH = 16384
_HIST = 20
_D = 32
_PW = _D // 2
_PACK = 8
_SCR_ROWS = 1000000 // _PACK
_SW = _PACK * _PW
_NBLK = 7813
_CHUNK = 128
_NC = 2
_NS = 16

_CP = pltpu.CompilerParams(use_tc_tiling_on_sc=True, needs_layout_passes=False)
_MESH = dict(core_axis_name="c", subcore_axis_name="s",
             num_cores=_NC, num_subcores=_NS)


@functools.cache
def _build_transpose():
    mesh = plsc.VectorSubcoreMesh(**_MESH)

    @functools.partial(
        pl.kernel,
        out_type=jax.ShapeDtypeStruct((_SCR_ROWS, _SW), jnp.int32),
        mesh=mesh,
        scratch_types=[
            pltpu.VMEM((2, _PW, 128), jnp.int32),
            pltpu.VMEM((_PW, 64), jnp.int32),
            pltpu.VMEM((2, _PW, 128), jnp.int32),
            pltpu.SemaphoreType.DMA,
            pltpu.SemaphoreType.DMA,
            pltpu.SemaphoreType.DMA,
            pltpu.SemaphoreType.DMA,
        ],
        compiler_params=_CP,
    )
    def transpose(table_hbm, tail_hbm, scr_hbm, blk_v, tblk_v, tp_v,
                  si0, si1, so0, so1):
        tblw = table_hbm.bitcast(jnp.int32)
        wid = lax.axis_index("s") * _NC + lax.axis_index("c")
        sin = (si0, si1)
        sout = (so0, so1)

        def blk_of(i):
            return wid + 32 * i

        def stage(blk, t):
            pltpu.async_copy(
                tblw.at[:, pl.ds(blk * 128, 128)], blk_v.at[t], sin[t]
            )

        def wait_in(t):
            pltpu.make_async_copy(
                tblw.at[:, pl.ds(0, 128)], blk_v.at[t], sin[t]
            ).wait()

        def flip(src_v, t, width):
            @pl.loop(0, width)
            def _(r):
                col = plsc.load_gather(
                    src_v,
                    [lax.iota(jnp.int32, _PW),
                     jnp.zeros((_PW,), jnp.int32) + r],
                )
                tp_v[t, r >> 3, pl.ds((r & 7) * _PW, _PW)] = col

        def store(blk, t):
            pltpu.async_copy(
                tp_v.at[t], scr_hbm.at[pl.ds(blk * 16, _PW), :], sout[t]
            )

        def wait_out(t):
            pltpu.make_async_copy(
                tp_v.at[t], scr_hbm.at[pl.ds(0, _PW), :], sout[t]
            ).wait()

        stage(blk_of(0), 0)
        stage(blk_of(1), 1)

        @pl.loop(0, 244, step=2)
        def _(i0):
            for t in range(2):
                i = i0 + t
                wait_in(t)

                @pl.when(i >= 2)
                def _():
                    wait_out(t)

                flip(blk_v.at[t], t, 128)

                @pl.when(i + 2 < 244)
                def _():
                    stage(blk_of(i + 2), t)

                store(blk_of(i), t)

        wait_out(0)
        wait_out(1)

        @pl.when(wid < 4)
        def _():
            blk = wid + 7808
            pltpu.sync_copy(tblw.at[:, pl.ds(blk * 128, 128)], blk_v.at[0])
            flip(blk_v.at[0], 0, 128)
            pltpu.sync_copy(tp_v.at[0], scr_hbm.at[pl.ds(blk * 16, _PW), :])

        @pl.when(wid == 4)
        def _():
            pltpu.sync_copy(tail_hbm.bitcast(jnp.int32), tblk_v)
            flip(tblk_v, 0, 64)
            pltpu.sync_copy(
                tp_v.at[0, pl.ds(0, 8), :],
                scr_hbm.at[pl.ds((_NBLK - 1) * 16, 8), :],
            )

    return transpose


@functools.cache
def _build_gather():
    nw = _NC * _NS
    b_per_w = _BATCH // nw
    n_sub = b_per_w // _CHUNK
    mesh = plsc.VectorSubcoreMesh(**_MESH)

    @functools.partial(
        pl.kernel,
        out_type=jax.ShapeDtypeStruct((_HIST, _D, _BATCH), jnp.bfloat16),
        mesh=mesh,
        scratch_types=[
            pltpu.VMEM((b_per_w,), jnp.int32),
            pltpu.VMEM((2, _CHUNK), jnp.int32),
            pltpu.VMEM((2, _CHUNK), jnp.int32),
            pltpu.VMEM((2, _CHUNK, _SW), jnp.int32),
            pltpu.VMEM((_PW, b_per_w), jnp.int32),
            pltpu.SemaphoreType.DMA,
            pltpu.SemaphoreType.DMA,
        ],
        compiler_params=_CP,
    )
    def gather(idx_hbm, scr_hbm, out_hbm, sidx_v, rid_v, off_v, pair_v,
               ob_v, sem0, sem1):
        wid = lax.axis_index("s") * _NC + lax.axis_index("c")
        b0 = wid * b_per_w
        sems = (sem0, sem1)

        def prep(c, tc):
            @pl.loop(0, _CHUNK, step=16)
            def _(i):
                v = sidx_v[pl.ds(c * _CHUNK + i, 16)]
                rid_v[tc, pl.ds(i, 16)] = v >> 3
                off_v[tc, pl.ds(i, 16)] = (v & 7) * _PW

        def issue(tc):
            pltpu.async_copy(scr_hbm.at[rid_v.at[tc]], pair_v.at[tc],
                             sems[tc])

        def drain(tc):
            pltpu.make_async_copy(
                scr_hbm.at[pl.ds(0, _CHUNK)], pair_v.at[tc], sems[tc]
            ).wait()

        def extract(c, tc):
            for kg in range(_CHUNK // 16):
                ks = lax.iota(jnp.int32, 16) + (kg * 16)
                offs = off_v[tc, pl.ds(kg * 16, 16)]

                @pl.loop(0, _PW)
                def _(p):
                    col = plsc.load_gather(pair_v.at[tc], [ks, offs + p])
                    ob_v[p, pl.ds(c * _CHUNK + kg * 16, 16)] = col

        @pl.loop(0, _HIST)
        def _(h):
            pltpu.sync_copy(idx_hbm.at[h, pl.ds(b0, b_per_w)], sidx_v)
            prep(0, 0)
            issue(0)
            for c in range(n_sub):
                tc = c % 2
                if c + 1 < n_sub:
                    prep(c + 1, 1 - tc)
                    issue(1 - tc)
                drain(tc)
                extract(c, tc)

            outw = out_hbm.at[h].bitcast(jnp.int32)
            pltpu.sync_copy(ob_v, outw.at[:, pl.ds(b0, b_per_w)])

    return gather


def kernel(inputs, embedding):
    table_t = embedding.T
    tail_t = table_t[:, (_NBLK - 1) * 128:]
    scr = _build_transpose()(table_t, tail_t)
    out = _build_gather()(inputs.T, scr)
    return out.transpose(2, 0, 1)
